# baseline (speedup 1.0000x reference)
import jax
import jax.numpy as jnp
from jax import lax
from jax.experimental import pallas as pl
from jax.experimental.pallas import tpu as pltpu
from jax.experimental.pallas import tpu_sc as plsc

_B = 64
_S = 2048
_D = 1024


_NC = 2
_BPC = _B // _NC


def _body(seq_hbm, tl_hbm, out_hbm, tl_s, sem):
    cid = lax.axis_index("c")
    base = cid * _BPC
    pltpu.sync_copy(tl_hbm, tl_s)
    copies = []
    for j in range(_BPC):
        t = tl_s[base + j]
        row = jnp.where(t == 0, _S - 1, t - 1)
        c = pltpu.make_async_copy(seq_hbm.at[base + j, row], out_hbm.at[base + j], sem)
        c.start()
        copies.append(c)
    for c in copies:
        c.wait()


@jax.jit
def kernel(sequence_embedding, token_len):
    mesh = plsc.ScalarSubcoreMesh(axis_name="c", num_cores=_NC)
    out = pl.kernel(
        _body,
        out_type=jax.ShapeDtypeStruct((_B, _D), jnp.float32),
        mesh=mesh,
        scratch_types=[
            pltpu.SMEM((_B,), jnp.int32),
            pltpu.SemaphoreType.DMA,
        ],
    )(sequence_embedding, token_len)
    return out


# speedup vs baseline: 1.0490x; 1.0490x over previous
import jax
import jax.numpy as jnp
from jax import lax
from jax.experimental import pallas as pl
from jax.experimental.pallas import tpu as pltpu
from jax.experimental.pallas import tpu_sc as plsc

_B = 64
_S = 2048
_D = 1024


def _body(seq_hbm, tl_hbm, out_hbm, tl_s, sem):
    pltpu.sync_copy(tl_hbm, tl_s)

    def fire(b, carry):
        t = tl_s[b]
        row = (t + (_S - 1)) & (_S - 1)
        pltpu.make_async_copy(seq_hbm.at[b, row], out_hbm.at[b], sem).start()
        return carry

    lax.fori_loop(0, _B, fire, 0)
    pltpu.make_async_copy(seq_hbm.at[0, pl.ds(0, _B), :], out_hbm, sem).wait()


@jax.jit
def kernel(sequence_embedding, token_len):
    mesh = plsc.ScalarSubcoreMesh(axis_name="c", num_cores=1)
    out = pl.kernel(
        _body,
        out_type=jax.ShapeDtypeStruct((_B, _D), jnp.float32),
        mesh=mesh,
        scratch_types=[
            pltpu.SMEM((_B,), jnp.int32),
            pltpu.SemaphoreType.DMA,
        ],
    )(sequence_embedding, token_len)
    return out
